# baseline (speedup 1.0000x reference)
import functools

import jax
import jax.numpy as jnp
from jax import lax
from jax.experimental import pallas as pl
from jax.experimental.pallas import tpu as pltpu
from jax.experimental.pallas import tpu_sc as plsc

N = 100000
E = 6400000
IN_DIM = 4
HID = 64
OUT = 128
G = 128

NC = 2
NS = 16
NW = NC * NS
DP = 8
OPW = 128
OPS = E // OPW
R = 16
OPS_W = 1568
OPS_LAST = OPS - 31 * OPS_W
RPT = 6256
NP = RPT * NS


def _edge_agg(x, edges, zeros):
    mesh = plsc.VectorSubcoreMesh(core_axis_name="c", subcore_axis_name="s")

    @functools.partial(
        pl.kernel,
        mesh=mesh,
        compiler_params=pltpu.CompilerParams(use_tc_tiling_on_sc=False),
        out_type=jax.ShapeDtypeStruct((NC, NP, DP), jnp.float32),
        scratch_types=[
            pltpu.VMEM((R, OPW), jnp.int32),
            pltpu.VMEM((R, OPW), jnp.int32),
            pltpu.VMEM((R, OPW, DP), jnp.float32),
            pltpu.VMEM_SHARED((NP, DP), jnp.float32),
            pltpu.SemaphoreType.DMA,
            pltpu.SemaphoreType.DMA,
            pltpu.SemaphoreType.DMA,
        ],
    )
    def k(x_hbm, edges_hbm, z_hbm, out_hbm, sidx, didx, rows, agg_sh,
          semi, semg, sems):
        cid = lax.axis_index("c")
        sid = lax.axis_index("s")
        wid = sid * NC + cid
        pltpu.sync_copy(
            z_hbm.at[pl.ds(sid * RPT, RPT)],
            agg_sh.at[pl.ds(sid * RPT, RPT)],
        )
        plsc.subcore_barrier()

        wop0 = wid * OPS_W
        nchunks = jnp.where(wid == NW - 1, OPS_LAST // R, OPS_W // R)

        def body(c, carry):
            cop = wop0 + c * R
            i1 = pltpu.async_copy(edges_hbm.at[0, pl.ds(cop, R)], sidx, semi)
            i2 = pltpu.async_copy(edges_hbm.at[1, pl.ds(cop, R)], didx, semi)
            i1.wait()
            gathers = [
                pltpu.async_copy(x_hbm.at[sidx.at[r]], rows.at[r], semg)
                for r in range(R)
            ]
            i2.wait()
            for g in gathers:
                g.wait()
            scatters = [
                pltpu.async_copy(rows.at[r], agg_sh.at[didx.at[r]], sems,
                                 add=True)
                for r in range(R)
            ]
            for s in scatters:
                s.wait()
            return carry

        lax.fori_loop(0, nchunks, body, 0)
        plsc.subcore_barrier()
        pltpu.sync_copy(
            agg_sh.at[pl.ds(sid * RPT, RPT)],
            out_hbm.at[cid, pl.ds(sid * RPT, RPT)],
        )

    return k(x, edges, zeros)


BN = 5000
STEPS = N // BN


def _mlp_pool_body(z_r, b_r, w1_r, b1_r, w2_r, b2_r, w3_r, b3_r,
                   out_r, acc_r):
    i = pl.program_id(0)

    @pl.when(i == 0)
    def _():
        acc_r[...] = jnp.zeros_like(acc_r)

    z = z_r[...]
    h = jnp.maximum(jnp.dot(z, w1_r[...], preferred_element_type=jnp.float32)
                    + b1_r[...], 0.0)
    h = jnp.dot(h, w2_r[...], preferred_element_type=jnp.float32) + b2_r[...]
    bb = b_r[0, 0, :]
    onehot = (bb[:, None] == lax.broadcasted_iota(jnp.int32, (BN, G), 1)
              ).astype(jnp.float32)
    hc = jnp.concatenate([h, jnp.ones((BN, 1), jnp.float32)], axis=1)
    acc_r[...] += lax.dot_general(onehot, hc, (((0,), (0,)), ((), ())),
                                  preferred_element_type=jnp.float32)

    @pl.when(i == STEPS - 1)
    def _():
        acc = acc_r[...]
        cnt = jnp.clip(acc[:, HID:HID + 1], 1.0, None)
        pooled = acc[:, :HID] / cnt
        out_r[...] = (jnp.dot(pooled, w3_r[...],
                              preferred_element_type=jnp.float32) + b3_r[...])


def _mlp_pool(z, batch3, W1, b1, W2, b2, W3, b3):
    return pl.pallas_call(
        _mlp_pool_body,
        grid=(STEPS,),
        in_specs=[
            pl.BlockSpec((BN, IN_DIM), lambda i: (i, 0)),
            pl.BlockSpec((1, 1, BN), lambda i: (i, 0, 0)),
            pl.BlockSpec((IN_DIM, HID), lambda i: (0, 0)),
            pl.BlockSpec((1, HID), lambda i: (0, 0)),
            pl.BlockSpec((HID, HID), lambda i: (0, 0)),
            pl.BlockSpec((1, HID), lambda i: (0, 0)),
            pl.BlockSpec((HID, OUT), lambda i: (0, 0)),
            pl.BlockSpec((1, OUT), lambda i: (0, 0)),
        ],
        out_specs=pl.BlockSpec((G, OUT), lambda i: (0, 0)),
        out_shape=jax.ShapeDtypeStruct((G, OUT), jnp.float32),
        scratch_shapes=[pltpu.VMEM((G, HID + 1), jnp.float32)],
    )(z, batch3, W1, b1[None], W2, b2[None], W3, b3[None])


def kernel(x, edge_index, batch, W1, b1, W2, b2, W3, b3):
    edges = edge_index.reshape(2, OPS, OPW)
    x16 = jnp.pad(x, ((0, 0), (0, DP - IN_DIM)))
    zeros = jnp.zeros((NP, DP), jnp.float32)
    partials = _edge_agg(x16, edges, zeros)
    z = x + partials[0, :N, :IN_DIM] + partials[1, :N, :IN_DIM]
    batch3 = batch.reshape(STEPS, 1, BN)
    return _mlp_pool(z, batch3, W1, b1, W2, b2, W3, b3)


# speedup vs baseline: 1.1321x; 1.1321x over previous
import functools

import jax
import jax.numpy as jnp
from jax import lax
from jax.experimental import pallas as pl
from jax.experimental.pallas import tpu as pltpu
from jax.experimental.pallas import tpu_sc as plsc

N = 100000
E = 6400000
IN_DIM = 4
HID = 64
OUT = 128
G = 128

NC = 2
NS = 16
NW = NC * NS
DP = 8
OPW = 128
OPS = E // OPW
R = 16
OPS_W = 1568
OPS_LAST = OPS - 31 * OPS_W
RPT = 6256
NP = RPT * NS


def _edge_agg(x, edges, zeros):
    mesh = plsc.VectorSubcoreMesh(core_axis_name="c", subcore_axis_name="s")

    @functools.partial(
        pl.kernel,
        mesh=mesh,
        compiler_params=pltpu.CompilerParams(use_tc_tiling_on_sc=False),
        out_type=jax.ShapeDtypeStruct((NC, NP, DP), jnp.float32),
        scratch_types=[
            pltpu.VMEM((2, R, OPW), jnp.int32),
            pltpu.VMEM((2, R, OPW), jnp.int32),
            pltpu.VMEM((2, R, OPW, DP), jnp.float32),
            pltpu.VMEM_SHARED((NP, DP), jnp.float32),
            pltpu.SemaphoreType.DMA,
            pltpu.SemaphoreType.DMA,
            pltpu.SemaphoreType.DMA,
        ],
    )
    def k(x_hbm, edges_hbm, z_hbm, out_hbm, sidx, didx, rows, agg_sh,
          semi, semg, sems):
        cid = lax.axis_index("c")
        sid = lax.axis_index("s")
        wid = sid * NC + cid
        pltpu.sync_copy(
            z_hbm.at[pl.ds(sid * RPT, RPT)],
            agg_sh.at[pl.ds(sid * RPT, RPT)],
        )
        plsc.subcore_barrier()

        wop0 = wid * OPS_W
        nchunks = jnp.where(wid == NW - 1, OPS_LAST // R, OPS_W // R)

        def drain(buf_b, sem):
            for r in range(R):
                pltpu.make_async_copy(
                    z_hbm.at[pl.ds(0, OPW)], rows.at[buf_b, r], sem,
                ).wait()

        def fire_gathers(b, c):
            cop = wop0 + c * R
            i1 = pltpu.async_copy(edges_hbm.at[0, pl.ds(cop, R)],
                                  sidx.at[b], semi)
            i2 = pltpu.async_copy(edges_hbm.at[1, pl.ds(cop, R)],
                                  didx.at[b], semi)
            i1.wait()
            i2.wait()
            for r in range(R):
                pltpu.async_copy(x_hbm.at[sidx.at[b, r]], rows.at[b, r], semg)

        def fire_scatters(b):
            for r in range(R):
                pltpu.async_copy(rows.at[b, r], agg_sh.at[didx.at[b, r]],
                                 sems, add=True)

        def body(c, carry):
            b = lax.rem(c, 2)
            pb = 1 - b

            @pl.when(c >= 2)
            def _():
                drain(b, sems)

            @pl.when(c >= 1)
            def _():
                drain(pb, semg)
                fire_scatters(pb)

            fire_gathers(b, c)
            return carry

        lax.fori_loop(0, nchunks, body, 0)
        lb = lax.rem(nchunks - 1, 2)

        @pl.when(nchunks >= 2)
        def _():
            drain(1 - lb, sems)
        drain(lb, semg)
        fire_scatters(lb)
        drain(lb, sems)
        plsc.subcore_barrier()
        pltpu.sync_copy(
            agg_sh.at[pl.ds(sid * RPT, RPT)],
            out_hbm.at[cid, pl.ds(sid * RPT, RPT)],
        )

    return k(x, edges, zeros)


BN = 2000
STEPS = N // BN


def _mlp_pool_body(x_r, a0_r, a1_r, b_r, w1_r, b1_r, w2_r, b2_r, w3_r, b3_r,
                   out_r, acc_r):
    i = pl.program_id(0)

    @pl.when(i == 0)
    def _():
        acc_r[...] = jnp.zeros_like(acc_r)

    z = x_r[...] + a0_r[0][:, :IN_DIM] + a1_r[0][:, :IN_DIM]
    h = jnp.maximum(jnp.dot(z, w1_r[...], preferred_element_type=jnp.float32)
                    + b1_r[...], 0.0)
    h = jnp.dot(h, w2_r[...], preferred_element_type=jnp.float32) + b2_r[...]
    bb = b_r[0, 0, :]
    onehot = (bb[:, None] == lax.broadcasted_iota(jnp.int32, (BN, G), 1)
              ).astype(jnp.float32)
    hc = jnp.concatenate([h, jnp.ones((BN, 1), jnp.float32)], axis=1)
    acc_r[...] += lax.dot_general(onehot, hc, (((0,), (0,)), ((), ())),
                                  preferred_element_type=jnp.float32)

    @pl.when(i == STEPS - 1)
    def _():
        acc = acc_r[...]
        cnt = jnp.clip(acc[:, HID:HID + 1], 1.0, None)
        pooled = acc[:, :HID] / cnt
        out_r[...] = (jnp.dot(pooled, w3_r[...],
                              preferred_element_type=jnp.float32) + b3_r[...])


def _mlp_pool(x, partials, batch3, W1, b1, W2, b2, W3, b3):
    return pl.pallas_call(
        _mlp_pool_body,
        grid=(STEPS,),
        in_specs=[
            pl.BlockSpec((BN, IN_DIM), lambda i: (i, 0)),
            pl.BlockSpec((1, BN, DP), lambda i: (0, i, 0)),
            pl.BlockSpec((1, BN, DP), lambda i: (1, i, 0)),
            pl.BlockSpec((1, 1, BN), lambda i: (i, 0, 0)),
            pl.BlockSpec((IN_DIM, HID), lambda i: (0, 0)),
            pl.BlockSpec((1, HID), lambda i: (0, 0)),
            pl.BlockSpec((HID, HID), lambda i: (0, 0)),
            pl.BlockSpec((1, HID), lambda i: (0, 0)),
            pl.BlockSpec((HID, OUT), lambda i: (0, 0)),
            pl.BlockSpec((1, OUT), lambda i: (0, 0)),
        ],
        out_specs=pl.BlockSpec((G, OUT), lambda i: (0, 0)),
        out_shape=jax.ShapeDtypeStruct((G, OUT), jnp.float32),
        scratch_shapes=[pltpu.VMEM((G, HID + 1), jnp.float32)],
    )(x, partials, partials, batch3, W1, b1[None], W2, b2[None], W3, b3[None])


def kernel(x, edge_index, batch, W1, b1, W2, b2, W3, b3):
    edges = edge_index.reshape(2, OPS, OPW)
    x16 = jnp.pad(x, ((0, 0), (0, DP - IN_DIM)))
    zeros = jnp.zeros((NP, DP), jnp.float32)
    partials = _edge_agg(x16, edges, zeros)
    batch3 = batch.reshape(STEPS, 1, BN)
    return _mlp_pool(x, partials, batch3, W1, b1, W2, b2, W3, b3)
